# baseline (speedup 1.0000x reference)
import functools

import jax
import jax.numpy as jnp
from jax import lax
from jax.experimental import pallas as pl
from jax.experimental.pallas import tpu as pltpu
from jax.experimental.pallas import tpu_sc as plsc

B = 16384
NUM_ASPECTS = 5
H1 = 128

_NC = 2
_NS = 16
_NW = _NC * _NS
_ROWS = 128
_COLS = 128
_RPW = _ROWS // _NW


def _sc_offsets_body(uid_hbm, iid_hbm, utab_hbm, itab_hbm, out_hbm,
                     uidx, iidx, urow, irow, comb, sem):
    wid = lax.axis_index("s") * _NC + lax.axis_index("c")
    base = wid * _RPW
    pltpu.sync_copy(uid_hbm.at[pl.ds(base, _RPW)], uidx)
    pltpu.sync_copy(iid_hbm.at[pl.ds(base, _RPW)], iidx)
    utab = utab_hbm.at[0]
    itab = itab_hbm.at[0]
    handles = []
    for j in range(_RPW):
        handles.append(pltpu.async_copy(utab.at[uidx.at[j]], urow.at[j], sem))
        handles.append(pltpu.async_copy(itab.at[iidx.at[j]], irow.at[j], sem))
    for h in handles:
        h.wait()
    for j in range(_RPW):
        for k in range(_COLS // 16):
            sl = pl.ds(k * 16, 16)
            comb[j, sl] = urow[j, sl] + irow[j, sl]
    pltpu.sync_copy(comb, out_hbm.at[pl.ds(base, _RPW)])


def _sc_offsets(uid2, iid2, utab2, itab2):
    mesh = plsc.VectorSubcoreMesh(core_axis_name="c", subcore_axis_name="s")
    kern = functools.partial(
        pl.kernel,
        mesh=mesh,
        out_type=jax.ShapeDtypeStruct((_ROWS, _COLS), jnp.float32),
        scratch_types=[
            pltpu.VMEM((_RPW, _COLS), jnp.int32),
            pltpu.VMEM((_RPW, _COLS), jnp.int32),
            pltpu.VMEM((_RPW, _COLS), jnp.float32),
            pltpu.VMEM((_RPW, _COLS), jnp.float32),
            pltpu.VMEM((_RPW, _COLS), jnp.float32),
            pltpu.SemaphoreType.DMA,
        ],
    )(_sc_offsets_body)
    return kern(uid2, iid2, utab2, itab2)


_RB = 1024
_NB = B // _RB


def _tc_body(g_ref, u_ref, v_ref, c_ref, o_ref):
    s = jnp.sum(u_ref[...] * v_ref[...], axis=(0, 2))
    o_ref[...] = s.reshape(_RB // _COLS, _COLS) + c_ref[...] + g_ref[0, 0]


def _tc_rating(u_t, v_t, comb, g2):
    return pl.pallas_call(
        _tc_body,
        grid=(_NB,),
        in_specs=[
            pl.BlockSpec(memory_space=pltpu.SMEM),
            pl.BlockSpec((NUM_ASPECTS, _RB, H1), lambda i: (0, i, 0)),
            pl.BlockSpec((NUM_ASPECTS, _RB, H1), lambda i: (0, i, 0)),
            pl.BlockSpec((_RB // _COLS, _COLS), lambda i: (i, 0)),
        ],
        out_specs=pl.BlockSpec((_RB // _COLS, _COLS), lambda i: (i, 0)),
        out_shape=jax.ShapeDtypeStruct((_ROWS, _COLS), jnp.float32),
    )(g2, u_t, v_t, comb)


def kernel(userAspRep, itemAspRep, batch_uid, batch_iid, user_offset,
           item_offset, global_offset):
    uid2 = batch_uid.reshape(_ROWS, _COLS)
    iid2 = batch_iid.reshape(_ROWS, _COLS)
    utab2 = user_offset.T
    itab2 = item_offset.T
    comb = _sc_offsets(uid2, iid2, utab2, itab2)

    u_t = userAspRep.transpose(1, 0, 2)
    v_t = itemAspRep.transpose(1, 0, 2)
    g2 = global_offset.reshape(1, 1)
    out = _tc_rating(u_t, v_t, comb, g2)
    return out.reshape(B, 1)


# speedup vs baseline: 1.1179x; 1.1179x over previous
import functools

import jax
import jax.numpy as jnp
from jax import lax
from jax.experimental import pallas as pl
from jax.experimental.pallas import tpu as pltpu
from jax.experimental.pallas import tpu_sc as plsc

B = 16384
NUM_ASPECTS = 5
H1 = 128

_NC = 2
_NS = 16
_NW = _NC * _NS
_ROWS = 128
_COLS = 128
_RPW = _ROWS // _NW


def _sc_offsets_body(uid_hbm, iid_hbm, utab_hbm, itab_hbm, out_hbm,
                     uidx, iidx, urow, irow, comb, sem):
    wid = lax.axis_index("s") * _NC + lax.axis_index("c")
    base = wid * _RPW
    pltpu.sync_copy(uid_hbm.at[pl.ds(base, _RPW)], uidx)
    pltpu.sync_copy(iid_hbm.at[pl.ds(base, _RPW)], iidx)
    utab = utab_hbm.at[0]
    itab = itab_hbm.at[0]
    handles = []
    for j in range(_RPW):
        handles.append(pltpu.async_copy(utab.at[uidx.at[j]], urow.at[j], sem))
        handles.append(pltpu.async_copy(itab.at[iidx.at[j]], irow.at[j], sem))
    for h in handles:
        h.wait()
    for j in range(_RPW):
        for k in range(_COLS // 16):
            sl = pl.ds(k * 16, 16)
            comb[j, sl] = urow[j, sl] + irow[j, sl]
    pltpu.sync_copy(comb, out_hbm.at[pl.ds(base, _RPW)])


def _sc_offsets(uid2, iid2, utab2, itab2):
    mesh = plsc.VectorSubcoreMesh(core_axis_name="c", subcore_axis_name="s")
    kern = functools.partial(
        pl.kernel,
        mesh=mesh,
        out_type=jax.ShapeDtypeStruct((_ROWS, _COLS), jnp.float32),
        scratch_types=[
            pltpu.VMEM((_RPW, _COLS), jnp.int32),
            pltpu.VMEM((_RPW, _COLS), jnp.int32),
            pltpu.VMEM((_RPW, _COLS), jnp.float32),
            pltpu.VMEM((_RPW, _COLS), jnp.float32),
            pltpu.VMEM((_RPW, _COLS), jnp.float32),
            pltpu.SemaphoreType.DMA,
        ],
    )(_sc_offsets_body)
    return kern(uid2, iid2, utab2, itab2)


_RB = 1024
_NB = B // _RB


def _tc_body(g_ref, u_ref, v_ref, o_ref):
    s = jnp.sum(u_ref[...] * v_ref[...], axis=(0, 2))
    o_ref[...] = s.reshape(_RB // _COLS, _COLS) + g_ref[0, 0]


def _tc_dots(u_t, v_t, g2):
    return pl.pallas_call(
        _tc_body,
        grid=(_NB,),
        in_specs=[
            pl.BlockSpec(memory_space=pltpu.SMEM),
            pl.BlockSpec((NUM_ASPECTS, _RB, H1), lambda i: (0, i, 0)),
            pl.BlockSpec((NUM_ASPECTS, _RB, H1), lambda i: (0, i, 0)),
        ],
        out_specs=pl.BlockSpec((_RB // _COLS, _COLS), lambda i: (i, 0)),
        out_shape=jax.ShapeDtypeStruct((_ROWS, _COLS), jnp.float32),
    )(g2, u_t, v_t)


def _add_body(a_ref, b_ref, o_ref):
    o_ref[...] = a_ref[...] + b_ref[...]


def _tc_add(a, b):
    return pl.pallas_call(
        _add_body,
        out_shape=jax.ShapeDtypeStruct((_ROWS, _COLS), jnp.float32),
    )(a, b)


def kernel(userAspRep, itemAspRep, batch_uid, batch_iid, user_offset,
           item_offset, global_offset):
    uid2 = batch_uid.reshape(_ROWS, _COLS)
    iid2 = batch_iid.reshape(_ROWS, _COLS)
    utab2 = user_offset.T
    itab2 = item_offset.T
    comb = _sc_offsets(uid2, iid2, utab2, itab2)

    u_t = userAspRep.transpose(1, 0, 2)
    v_t = itemAspRep.transpose(1, 0, 2)
    g2 = global_offset.reshape(1, 1)
    dots = _tc_dots(u_t, v_t, g2)
    out = _tc_add(dots, comb)
    return out.reshape(B, 1)
